# baseline (speedup 1.0000x reference)
import functools
import jax
import jax.numpy as jnp
from jax import lax
from jax.experimental import pallas as pl
from jax.experimental.pallas import tpu as pltpu
from jax.experimental.pallas import tpu_sc as _plsc

N = 10000
T_ENC = 12
TRG_LEN = 12
H = 128
OUT = 2
E_RAW = 160000
E2 = E_RAW + N
SRC_PAD = 170048


def _lstm_body(x_ref, w1_ref, b1_ref, whhT_ref, h0_ref, c0_ref,
               ys_ref, hout_ref, cout_ref, xg_scr, h_scr, c_scr,
               *, chunk, B, IN, ng):
    @pl.when(pl.program_id(0) == 0)
    def _():
        h_scr[...] = h0_ref[...]
        c_scr[...] = c0_ref[...]

    xg = jnp.dot(x_ref[...].reshape(chunk * B, IN), w1_ref[...],
                 preferred_element_type=jnp.float32) + b1_ref[...]
    xg_scr[...] = xg.reshape(chunk, B, 4 * H)

    def step(i, carry):
        h, c = carry
        g = xg_scr[i] + jnp.dot(h, whhT_ref[...],
                                preferred_element_type=jnp.float32)
        ig = jax.nn.sigmoid(g[:, 0 * H:1 * H])
        fg = jax.nn.sigmoid(g[:, 1 * H:2 * H])
        gg = jnp.tanh(g[:, 2 * H:3 * H])
        og = jax.nn.sigmoid(g[:, 3 * H:4 * H])
        c = fg * c + ig * gg
        h = og * jnp.tanh(c)
        ys_ref[i] = h
        return (h, c)

    h, c = lax.fori_loop(0, chunk, step, (h_scr[...], c_scr[...]))
    h_scr[...] = h
    c_scr[...] = c

    @pl.when(pl.program_id(0) == ng - 1)
    def _():
        hout_ref[...] = h
        cout_ref[...] = c


def _lstm_scan(x, w1, b1, whhT, h0, c0, chunk):
    n, B, IN = x.shape
    assert n % chunk == 0
    ng = n // chunk
    body = functools.partial(_lstm_body, chunk=chunk, B=B, IN=IN, ng=ng)
    return pl.pallas_call(
        body,
        grid=(ng,),
        in_specs=[
            pl.BlockSpec((chunk, B, IN), lambda g: (g, 0, 0)),
            pl.BlockSpec((IN, 4 * H), lambda g: (0, 0)),
            pl.BlockSpec((1, 4 * H), lambda g: (0, 0)),
            pl.BlockSpec((H, 4 * H), lambda g: (0, 0)),
            pl.BlockSpec((B, H), lambda g: (0, 0)),
            pl.BlockSpec((B, H), lambda g: (0, 0)),
        ],
        out_specs=[
            pl.BlockSpec((chunk, B, H), lambda g: (g, 0, 0)),
            pl.BlockSpec((B, H), lambda g: (0, 0)),
            pl.BlockSpec((B, H), lambda g: (0, 0)),
        ],
        out_shape=[
            jax.ShapeDtypeStruct((n, B, H), jnp.float32),
            jax.ShapeDtypeStruct((B, H), jnp.float32),
            jax.ShapeDtypeStruct((B, H), jnp.float32),
        ],
        scratch_shapes=[
            pltpu.VMEM((chunk, B, 4 * H), jnp.float32),
            pltpu.VMEM((B, H), jnp.float32),
            pltpu.VMEM((B, H), jnp.float32),
        ],
    )(x, w1, b1.reshape(1, 4 * H), whhT, h0, c0)


def _mm2_body(a_ref, w1_ref, b1_ref, w2_ref, b2_ref, o1_ref, o2_ref):
    a = a_ref[...]
    o1_ref[...] = jnp.dot(a, w1_ref[...],
                          preferred_element_type=jnp.float32) + b1_ref[...]
    o2_ref[...] = jnp.dot(a, w2_ref[...],
                          preferred_element_type=jnp.float32) + b2_ref[...]


def _mm2(a, w1, b1, w2, b2, tm):
    m, k = a.shape
    d1 = w1.shape[1]
    d2 = w2.shape[1]
    assert m % tm == 0
    return pl.pallas_call(
        _mm2_body,
        grid=(m // tm,),
        in_specs=[
            pl.BlockSpec((tm, k), lambda g: (g, 0)),
            pl.BlockSpec((k, d1), lambda g: (0, 0)),
            pl.BlockSpec((1, d1), lambda g: (0, 0)),
            pl.BlockSpec((k, d2), lambda g: (0, 0)),
            pl.BlockSpec((1, d2), lambda g: (0, 0)),
        ],
        out_specs=[
            pl.BlockSpec((tm, d1), lambda g: (g, 0)),
            pl.BlockSpec((tm, d2), lambda g: (g, 0)),
        ],
        out_shape=[
            jax.ShapeDtypeStruct((m, d1), jnp.float32),
            jax.ShapeDtypeStruct((m, d2), jnp.float32),
        ],
    )(a, w1, b1.reshape(1, d1), w2, b2.reshape(1, d2))


def _mm_body(a_ref, w_ref, b_ref, o_ref):
    o_ref[...] = jnp.dot(a_ref[...], w_ref[...],
                         preferred_element_type=jnp.float32) + b_ref[...]


def _mm(a, w, b, tm):
    m, k = a.shape
    d = w.shape[1]
    assert m % tm == 0
    return pl.pallas_call(
        _mm_body,
        grid=(m // tm,),
        in_specs=[
            pl.BlockSpec((tm, k), lambda g: (g, 0)),
            pl.BlockSpec((k, d), lambda g: (0, 0)),
            pl.BlockSpec((1, d), lambda g: (0, 0)),
        ],
        out_specs=pl.BlockSpec((tm, d), lambda g: (g, 0)),
        out_shape=jax.ShapeDtypeStruct((m, d), jnp.float32),
    )(a, w, b.reshape(1, d))


def _gat_edge_scaffold(hs, hd, attn, src, dst, heads, T):
    outs = []
    for t in range(T):
        hst = hs[t * N:(t + 1) * N].reshape(N, heads, H)
        hdt = hd[t * N:(t + 1) * N].reshape(N, heads, H)
        e = jax.nn.leaky_relu(hst[src] + hdt[dst], 0.2)
        logits = jnp.sum(e * attn[None], axis=-1)
        m = jax.ops.segment_max(logits, dst, num_segments=N)
        ex = jnp.exp(logits - m[dst])
        s = jax.ops.segment_sum(ex, dst, num_segments=N)
        alpha = ex / s[dst]
        rst = jax.ops.segment_sum(hst[src] * alpha[..., None], dst,
                                  num_segments=N)
        outs.append(rst.reshape(N, heads * H))
    return jnp.concatenate(outs, axis=0)


NSEG_PAD = 10024
NOWN_PAD = 48
NEG_BIG = -1e30


def _vsum(v):
    acc = v[0]
    for i in range(1, 16):
        acc = acc + v[i]
    return acc


def _vmax(v):
    acc = v[0]
    for i in range(1, 16):
        acc = jnp.maximum(acc, v[i])
    return acc


def _gat_sc_body(hs_hbm, hd_hbm, src_hbm, seg_hbm, own_hbm, attn_hbm,
                 rst_hbm, segst_v, own_v, attn_v, hd_v, idx_v, gidx_v,
                 rows_v, acc_v, sems, *, T, HEADS):
    D = HEADS * H
    NG = D // 16
    i32 = jnp.int32
    f32 = jnp.float32
    plsc = _plsc

    wid = lax.axis_index("s") * 2 + lax.axis_index("c")
    pltpu.sync_copy(seg_hbm, segst_v)
    pltpu.sync_copy(own_hbm, own_v)
    pltpu.sync_copy(attn_hbm, attn_v)
    ov = own_v[pl.ds(wid, 16)]
    d_lo = ov[0]
    d_hi = ov[1]

    lane = lax.broadcasted_iota(i32, (16,), 0)
    zeros16 = jnp.zeros((16,), f32)

    for b in range(2):
        gidx_v[b, :] = jnp.zeros((16,), i32)
        pltpu.async_copy(hs_hbm.at[gidx_v.at[b]], rows_v.at[b],
                         sems.at[b]).wait()

    def t_body(t, _tc):
        tn = t * N

        def node_body(d, _carry):
            kv = segst_v[pl.ds(d, 16)]
            k0 = kv[0]
            k1 = kv[1]
            pltpu.sync_copy(hd_hbm.at[pl.ds(tn + d, 1)], hd_v)
            for g in range(NG):
                acc_v[0, pl.ds(g * 16, 16)] = zeros16
            c0 = k0 - lax.rem(k0, 8)
            nch = lax.div(k1 - c0 + 15, 16)

            def start_load(ch, b):
                e0 = pl.multiple_of(c0 + ch * 16, 8)
                pltpu.sync_copy(src_hbm.at[pl.ds(e0, 16)], idx_v.at[b])
                gidx_v[b, :] = idx_v[b, :] + tn
                pltpu.async_copy(hs_hbm.at[gidx_v.at[b]], rows_v.at[b],
                                 sems.at[b])

            def compute_chunk(ch, b, carry):
                m_st, s_st = carry

                @pl.when(ch < nch)
                def _():
                    pltpu.make_async_copy(hs_hbm.at[gidx_v.at[b]],
                                          rows_v.at[b], sems.at[b]).wait()
                e0 = c0 + ch * 16
                elane = e0 + lane
                valid = (elane >= k0) & (elane < k1)

                m_new = []
                s_new = []
                w_list = []
                scale_list = []
                for h in range(HEADS):
                    hdg = [hd_v[0, pl.ds(h * H + g * 16, 16)]
                           for g in range(8)]
                    atg = [attn_v[0, pl.ds(h * H + g * 16, 16)]
                           for g in range(8)]
                    l_h = zeros16
                    for e in range(16):
                        a16 = zeros16
                        for g in range(8):
                            row = rows_v[b, e, pl.ds(h * H + g * 16, 16)]
                            x = row + hdg[g]
                            a16 = a16 + jnp.maximum(x, 0.2 * x) * atg[g]
                        l_h = jnp.where(lane == e,
                                        jnp.full((16,), _vsum(a16), f32),
                                        l_h)
                    l_h = jnp.where(valid, l_h, NEG_BIG)
                    cm = _vmax(l_h)
                    mh = jnp.maximum(m_st[h], jnp.full((16,), cm, f32))
                    w_h = jnp.exp(l_h - mh)
                    scale = jnp.exp(m_st[h] - mh)
                    sh = s_st[h] * scale + jnp.full((16,), _vsum(w_h), f32)
                    m_new.append(mh)
                    s_new.append(sh)
                    w_list.append(w_h)
                    scale_list.append(scale)

                @pl.when(ch + 2 < nch)
                def _():
                    start_load(ch + 2, b)

                for g in range(NG):
                    h = g // 8
                    a = acc_v[0, pl.ds(g * 16, 16)] * scale_list[h]
                    for e in range(16):
                        a = a + w_list[h][e] * rows_v[b, e,
                                                      pl.ds(g * 16, 16)]
                    acc_v[0, pl.ds(g * 16, 16)] = a
                return (tuple(m_new), tuple(s_new))

            start_load(0, 0)

            @pl.when(1 < nch)
            def _():
                start_load(1, 1)

            def pair_body(i, carry):
                for b in range(2):
                    carry = compute_chunk(i * 2 + b, b, carry)
                return carry

            init = (tuple(jnp.full((16,), NEG_BIG, f32)
                          for _ in range(HEADS)),
                    tuple(jnp.zeros((16,), f32) for _ in range(HEADS)))
            m_st, s_st = lax.fori_loop(0, lax.div(nch + 1, 2), pair_body,
                                       init)
            for h in range(HEADS):
                inv = 1.0 / s_st[h]
                for g in range(8 * h, 8 * h + 8):
                    acc_v[0, pl.ds(g * 16, 16)] = (
                        acc_v[0, pl.ds(g * 16, 16)] * inv)
            pltpu.sync_copy(acc_v, rst_hbm.at[pl.ds(tn + d, 1)])
            return 0

        lax.fori_loop(d_lo, d_hi, node_body, 0)
        return 0

    lax.fori_loop(0, T, t_body, 0)


def _gat_sc(hs, hd, attn, src_pad, seg_pad, own, T, HEADS):
    D = HEADS * H
    mesh = _plsc.VectorSubcoreMesh(core_axis_name="c", subcore_axis_name="s")
    body = functools.partial(_gat_sc_body, T=T, HEADS=HEADS)
    return pl.kernel(
        body,
        mesh=mesh,
        out_type=jax.ShapeDtypeStruct((T * N, D), jnp.float32),
        scratch_types=[
            pltpu.VMEM((NSEG_PAD,), jnp.int32),
            pltpu.VMEM((NOWN_PAD,), jnp.int32),
            pltpu.VMEM((1, D), jnp.float32),
            pltpu.VMEM((1, D), jnp.float32),
            pltpu.VMEM((2, 16), jnp.int32),
            pltpu.VMEM((2, 16), jnp.int32),
            pltpu.VMEM((2, 16, D), jnp.float32),
            pltpu.VMEM((1, D), jnp.float32),
            pltpu.SemaphoreType.DMA((2,)),
        ],
    )(hs, hd, src_pad, seg_pad, own, attn.reshape(1, D))


def kernel(src1, src2, edge_index, enc, dec):
    f32 = jnp.float32

    loops = jnp.arange(N, dtype=edge_index.dtype)
    src_e = jnp.concatenate([edge_index[0], loops])
    dst_e = jnp.concatenate([edge_index[1], loops])
    perm = jnp.argsort(dst_e)
    dst_s = dst_e[perm]
    src_s = src_e[perm]
    seg = jnp.searchsorted(dst_s, jnp.arange(N + 1, dtype=jnp.int32)
                           ).astype(jnp.int32)
    seg_pad = jnp.concatenate(
        [seg, jnp.full((NSEG_PAD - (N + 1),), E2, jnp.int32)])
    src_pad = jnp.concatenate(
        [src_s, jnp.zeros((SRC_PAD - E2,), jnp.int32)])
    epw = (E2 + 31) // 32
    own = jnp.searchsorted(seg[:N],
                           jnp.arange(33, dtype=jnp.int32) * epw,
                           side='left').astype(jnp.int32)
    own_pad = jnp.concatenate([own, jnp.full((NOWN_PAD - 33,), N, jnp.int32)])

    def lstm_weights(p):
        w1 = p["pre_W"] @ p["Wih"].T
        b1 = p["pre_b"] @ p["Wih"].T + p["bih"] + p["bhh"]
        return w1.astype(f32), b1.astype(f32), p["Whh"].T.astype(f32)

    enc_w1, enc_b1, enc_whhT = lstm_weights(enc)
    dec_w1, dec_b1, dec_whhT = lstm_weights(dec)

    z = jnp.zeros((T_ENC, H), f32)
    ys, h_enc, c_enc = _lstm_scan(src1, enc_w1, enc_b1, enc_whhT, z, z,
                                  chunk=250)

    ysT = jnp.transpose(ys, (1, 0, 2)).reshape(T_ENC * N, H)
    hs_e, hd_e = _mm2(ysT, enc["gsrc_W"], enc["gsrc_b"],
                      enc["gdst_W"], enc["gdst_b"], tm=400)
    HEADS_E = 4
    rst_e = _gat_sc(hs_e, hd_e, enc["attn"], src_pad, seg_pad, own_pad,
                    T_ENC, HEADS_E)
    fcW_e = jnp.tile(enc["fc_W"], (HEADS_E, 1)) / HEADS_E
    fcb_e = enc["fc_b"] + (enc["gbias"].reshape(HEADS_E, H).mean(0)
                           @ enc["fc_W"])
    enc_out_T = _mm(rst_e, fcW_e, fcb_e, tm=400)
    enc_out = jnp.transpose(enc_out_T.reshape(T_ENC, N, OUT), (1, 0, 2))

    fcW_d = dec["fc_W"]
    fcb_d = dec["fc_b"] + dec["gbias"] @ dec["fc_W"]

    hidden = h_enc[0:1, :]
    cell = c_enc[0:1, :]
    current = enc_out[:, T_ENC - 1, :]
    outs = []
    for t in range(TRG_LEN):
        inp = jnp.concatenate([current, src2[:, t, :]], axis=1)
        ys_d, hidden, cell = _lstm_scan(inp[:, None, :], dec_w1, dec_b1,
                                        dec_whhT, hidden, cell, chunk=500)
        feat = ys_d[:, 0, :]
        hs_d, hd_d = _mm2(feat, dec["gsrc_W"], dec["gsrc_b"],
                          dec["gdst_W"], dec["gdst_b"], tm=400)
        rst_d = _gat_sc(hs_d, hd_d, dec["attn"], src_pad, seg_pad,
                        own_pad, 1, 1)
        o = _mm(rst_d, fcW_d, fcb_d, tm=400)
        outs.append(o)
        current = o

    outputs = jnp.stack(outs, axis=1)
    prec = enc_out[:, 0:1, :]
    return jnp.concatenate([prec, outputs], axis=1)


# speedup vs baseline: 1.0668x; 1.0668x over previous
import functools
import jax
import jax.numpy as jnp
from jax import lax
from jax.experimental import pallas as pl
from jax.experimental.pallas import tpu as pltpu
from jax.experimental.pallas import tpu_sc as _plsc

N = 10000
T_ENC = 12
TRG_LEN = 12
H = 128
OUT = 2
E_RAW = 160000
E2 = E_RAW + N
SRC_PAD = 170048


def _lstm_body(x_ref, w1_ref, b1_ref, whhT_ref, h0_ref, c0_ref,
               ys_ref, hout_ref, cout_ref, xg_scr, h_scr, c_scr,
               *, chunk, B, IN, ng):
    @pl.when(pl.program_id(0) == 0)
    def _():
        h_scr[...] = h0_ref[...]
        c_scr[...] = c0_ref[...]

    xg = jnp.dot(x_ref[...].reshape(chunk * B, IN), w1_ref[...],
                 preferred_element_type=jnp.float32) + b1_ref[...]
    xg_scr[...] = xg.reshape(chunk, B, 4 * H)

    def step(i, carry):
        h, c = carry
        g = xg_scr[i] + jnp.dot(h, whhT_ref[...],
                                preferred_element_type=jnp.float32)
        ig = jax.nn.sigmoid(g[:, 0 * H:1 * H])
        fg = jax.nn.sigmoid(g[:, 1 * H:2 * H])
        gg = jnp.tanh(g[:, 2 * H:3 * H])
        og = jax.nn.sigmoid(g[:, 3 * H:4 * H])
        c = fg * c + ig * gg
        h = og * jnp.tanh(c)
        ys_ref[i] = h
        return (h, c)

    h, c = lax.fori_loop(0, chunk, step, (h_scr[...], c_scr[...]))
    h_scr[...] = h
    c_scr[...] = c

    @pl.when(pl.program_id(0) == ng - 1)
    def _():
        hout_ref[...] = h
        cout_ref[...] = c


def _lstm_scan(x, w1, b1, whhT, h0, c0, chunk):
    n, B, IN = x.shape
    assert n % chunk == 0
    ng = n // chunk
    body = functools.partial(_lstm_body, chunk=chunk, B=B, IN=IN, ng=ng)
    return pl.pallas_call(
        body,
        grid=(ng,),
        in_specs=[
            pl.BlockSpec((chunk, B, IN), lambda g: (g, 0, 0)),
            pl.BlockSpec((IN, 4 * H), lambda g: (0, 0)),
            pl.BlockSpec((1, 4 * H), lambda g: (0, 0)),
            pl.BlockSpec((H, 4 * H), lambda g: (0, 0)),
            pl.BlockSpec((B, H), lambda g: (0, 0)),
            pl.BlockSpec((B, H), lambda g: (0, 0)),
        ],
        out_specs=[
            pl.BlockSpec((chunk, B, H), lambda g: (g, 0, 0)),
            pl.BlockSpec((B, H), lambda g: (0, 0)),
            pl.BlockSpec((B, H), lambda g: (0, 0)),
        ],
        out_shape=[
            jax.ShapeDtypeStruct((n, B, H), jnp.float32),
            jax.ShapeDtypeStruct((B, H), jnp.float32),
            jax.ShapeDtypeStruct((B, H), jnp.float32),
        ],
        scratch_shapes=[
            pltpu.VMEM((chunk, B, 4 * H), jnp.float32),
            pltpu.VMEM((B, H), jnp.float32),
            pltpu.VMEM((B, H), jnp.float32),
        ],
    )(x, w1, b1.reshape(1, 4 * H), whhT, h0, c0)


def _mm2_body(a_ref, w1_ref, b1_ref, w2_ref, b2_ref, o1_ref, o2_ref):
    a = a_ref[...]
    o1_ref[...] = jnp.dot(a, w1_ref[...],
                          preferred_element_type=jnp.float32) + b1_ref[...]
    o2_ref[...] = jnp.dot(a, w2_ref[...],
                          preferred_element_type=jnp.float32) + b2_ref[...]


def _mm2(a, w1, b1, w2, b2, tm):
    m, k = a.shape
    d1 = w1.shape[1]
    d2 = w2.shape[1]
    assert m % tm == 0
    return pl.pallas_call(
        _mm2_body,
        grid=(m // tm,),
        in_specs=[
            pl.BlockSpec((tm, k), lambda g: (g, 0)),
            pl.BlockSpec((k, d1), lambda g: (0, 0)),
            pl.BlockSpec((1, d1), lambda g: (0, 0)),
            pl.BlockSpec((k, d2), lambda g: (0, 0)),
            pl.BlockSpec((1, d2), lambda g: (0, 0)),
        ],
        out_specs=[
            pl.BlockSpec((tm, d1), lambda g: (g, 0)),
            pl.BlockSpec((tm, d2), lambda g: (g, 0)),
        ],
        out_shape=[
            jax.ShapeDtypeStruct((m, d1), jnp.float32),
            jax.ShapeDtypeStruct((m, d2), jnp.float32),
        ],
    )(a, w1, b1.reshape(1, d1), w2, b2.reshape(1, d2))


def _mm_body(a_ref, w_ref, b_ref, o_ref):
    o_ref[...] = jnp.dot(a_ref[...], w_ref[...],
                         preferred_element_type=jnp.float32) + b_ref[...]


def _mm(a, w, b, tm):
    m, k = a.shape
    d = w.shape[1]
    assert m % tm == 0
    return pl.pallas_call(
        _mm_body,
        grid=(m // tm,),
        in_specs=[
            pl.BlockSpec((tm, k), lambda g: (g, 0)),
            pl.BlockSpec((k, d), lambda g: (0, 0)),
            pl.BlockSpec((1, d), lambda g: (0, 0)),
        ],
        out_specs=pl.BlockSpec((tm, d), lambda g: (g, 0)),
        out_shape=jax.ShapeDtypeStruct((m, d), jnp.float32),
    )(a, w, b.reshape(1, d))


def _gat_edge_scaffold(hs, hd, attn, src, dst, heads, T):
    outs = []
    for t in range(T):
        hst = hs[t * N:(t + 1) * N].reshape(N, heads, H)
        hdt = hd[t * N:(t + 1) * N].reshape(N, heads, H)
        e = jax.nn.leaky_relu(hst[src] + hdt[dst], 0.2)
        logits = jnp.sum(e * attn[None], axis=-1)
        m = jax.ops.segment_max(logits, dst, num_segments=N)
        ex = jnp.exp(logits - m[dst])
        s = jax.ops.segment_sum(ex, dst, num_segments=N)
        alpha = ex / s[dst]
        rst = jax.ops.segment_sum(hst[src] * alpha[..., None], dst,
                                  num_segments=N)
        outs.append(rst.reshape(N, heads * H))
    return jnp.concatenate(outs, axis=0)


NSEG_PAD = 10024
NOWN_PAD = 48
NEG_BIG = -1e30


def _vsum(v):
    acc = v[0]
    for i in range(1, 16):
        acc = acc + v[i]
    return acc


def _vmax(v):
    acc = v[0]
    for i in range(1, 16):
        acc = jnp.maximum(acc, v[i])
    return acc


def _gat_sc_body(hs_hbm, hd_hbm, src_hbm, seg_hbm, own_hbm, attn_hbm,
                 rst_hbm, segst_v, own_v, attn_v, hd_v, idx_v, gidx_v,
                 rows_v, acc_v, sems, *, T, HEADS):
    D = HEADS * H
    NG = D // 16
    i32 = jnp.int32
    f32 = jnp.float32
    plsc = _plsc

    wid = lax.axis_index("s") * 2 + lax.axis_index("c")
    pltpu.sync_copy(seg_hbm, segst_v)
    pltpu.sync_copy(own_hbm, own_v)
    pltpu.sync_copy(attn_hbm, attn_v)
    ov = own_v[pl.ds(wid, 16)]
    d_lo = ov[0]
    d_hi = ov[1]

    lane = lax.broadcasted_iota(i32, (16,), 0)
    zeros16 = jnp.zeros((16,), f32)

    for b in range(2):
        gidx_v[b, :] = jnp.zeros((16,), i32)
        pltpu.async_copy(hs_hbm.at[gidx_v.at[b]], rows_v.at[b],
                         sems.at[b]).wait()

    def t_body(t, _tc):
        tn = t * N

        def node_body(d, _carry):
            kv = segst_v[pl.ds(d, 16)]
            k0 = kv[0]
            k1 = kv[1]
            pltpu.sync_copy(hd_hbm.at[pl.ds(tn + d, 1)], hd_v)
            for g in range(NG):
                acc_v[0, pl.ds(g * 16, 16)] = zeros16
            c0 = k0 - lax.rem(k0, 8)
            nch = lax.div(k1 - c0 + 15, 16)

            def start_load(ch, b):
                e0 = pl.multiple_of(c0 + ch * 16, 8)
                pltpu.sync_copy(src_hbm.at[pl.ds(e0, 16)], idx_v.at[b])
                gidx_v[b, :] = idx_v[b, :] + tn
                pltpu.async_copy(hs_hbm.at[gidx_v.at[b]], rows_v.at[b],
                                 sems.at[b])

            def compute_chunk(ch, b, carry):
                m_st, s_st = carry
                start_load(ch, b)
                pltpu.make_async_copy(hs_hbm.at[gidx_v.at[b]],
                                      rows_v.at[b], sems.at[b]).wait()
                e0 = c0 + ch * 16
                elane = e0 + lane
                valid = (elane >= k0) & (elane < k1)

                m_new = []
                s_new = []
                w_list = []
                scale_list = []
                for h in range(HEADS):
                    hdg = [hd_v[0, pl.ds(h * H + g * 16, 16)]
                           for g in range(8)]
                    atg = [attn_v[0, pl.ds(h * H + g * 16, 16)]
                           for g in range(8)]
                    l_h = zeros16
                    for e in range(16):
                        a16 = zeros16
                        for g in range(8):
                            row = rows_v[b, e, pl.ds(h * H + g * 16, 16)]
                            x = row + hdg[g]
                            a16 = a16 + jnp.maximum(x, 0.2 * x) * atg[g]
                        l_h = jnp.where(lane == e,
                                        jnp.full((16,), _vsum(a16), f32),
                                        l_h)
                    l_h = jnp.where(valid, l_h, NEG_BIG)
                    cm = _vmax(l_h)
                    mh = jnp.maximum(m_st[h], jnp.full((16,), cm, f32))
                    w_h = jnp.exp(l_h - mh)
                    scale = jnp.exp(m_st[h] - mh)
                    sh = s_st[h] * scale + jnp.full((16,), _vsum(w_h), f32)
                    m_new.append(mh)
                    s_new.append(sh)
                    w_list.append(w_h)
                    scale_list.append(scale)

                for g in range(NG):
                    h = g // 8
                    a = acc_v[0, pl.ds(g * 16, 16)] * scale_list[h]
                    for e in range(16):
                        a = a + w_list[h][e] * rows_v[b, e,
                                                      pl.ds(g * 16, 16)]
                    acc_v[0, pl.ds(g * 16, 16)] = a
                return (tuple(m_new), tuple(s_new))

            def chunk_body(ch, carry):
                return compute_chunk(ch, 0, carry)

            init = (tuple(jnp.full((16,), NEG_BIG, f32)
                          for _ in range(HEADS)),
                    tuple(jnp.zeros((16,), f32) for _ in range(HEADS)))
            m_st, s_st = lax.fori_loop(0, nch, chunk_body, init)
            for h in range(HEADS):
                inv = 1.0 / s_st[h]
                for g in range(8 * h, 8 * h + 8):
                    acc_v[0, pl.ds(g * 16, 16)] = (
                        acc_v[0, pl.ds(g * 16, 16)] * inv)
            pltpu.sync_copy(acc_v, rst_hbm.at[pl.ds(tn + d, 1)])
            return 0

        lax.fori_loop(d_lo, d_hi, node_body, 0)
        return 0

    lax.fori_loop(0, T, t_body, 0)


def _gat_sc(hs, hd, attn, src_pad, seg_pad, own, T, HEADS):
    D = HEADS * H
    mesh = _plsc.VectorSubcoreMesh(core_axis_name="c", subcore_axis_name="s")
    body = functools.partial(_gat_sc_body, T=T, HEADS=HEADS)
    return pl.kernel(
        body,
        mesh=mesh,
        out_type=jax.ShapeDtypeStruct((T * N, D), jnp.float32),
        scratch_types=[
            pltpu.VMEM((NSEG_PAD,), jnp.int32),
            pltpu.VMEM((NOWN_PAD,), jnp.int32),
            pltpu.VMEM((1, D), jnp.float32),
            pltpu.VMEM((1, D), jnp.float32),
            pltpu.VMEM((2, 16), jnp.int32),
            pltpu.VMEM((2, 16), jnp.int32),
            pltpu.VMEM((2, 16, D), jnp.float32),
            pltpu.VMEM((1, D), jnp.float32),
            pltpu.SemaphoreType.DMA((2,)),
        ],
    )(hs, hd, src_pad, seg_pad, own, attn.reshape(1, D))


def kernel(src1, src2, edge_index, enc, dec):
    f32 = jnp.float32

    loops = jnp.arange(N, dtype=edge_index.dtype)
    src_e = jnp.concatenate([edge_index[0], loops])
    dst_e = jnp.concatenate([edge_index[1], loops])
    perm = jnp.argsort(dst_e)
    dst_s = dst_e[perm]
    src_s = src_e[perm]
    seg = jnp.searchsorted(dst_s, jnp.arange(N + 1, dtype=jnp.int32)
                           ).astype(jnp.int32)
    seg_pad = jnp.concatenate(
        [seg, jnp.full((NSEG_PAD - (N + 1),), E2, jnp.int32)])
    src_pad = jnp.concatenate(
        [src_s, jnp.zeros((SRC_PAD - E2,), jnp.int32)])
    epw = (E2 + 31) // 32
    own = jnp.searchsorted(seg[:N],
                           jnp.arange(33, dtype=jnp.int32) * epw,
                           side='left').astype(jnp.int32)
    own_pad = jnp.concatenate([own, jnp.full((NOWN_PAD - 33,), N, jnp.int32)])

    def lstm_weights(p):
        w1 = p["pre_W"] @ p["Wih"].T
        b1 = p["pre_b"] @ p["Wih"].T + p["bih"] + p["bhh"]
        return w1.astype(f32), b1.astype(f32), p["Whh"].T.astype(f32)

    enc_w1, enc_b1, enc_whhT = lstm_weights(enc)
    dec_w1, dec_b1, dec_whhT = lstm_weights(dec)

    z = jnp.zeros((T_ENC, H), f32)
    ys, h_enc, c_enc = _lstm_scan(src1, enc_w1, enc_b1, enc_whhT, z, z,
                                  chunk=250)

    ysT = jnp.transpose(ys, (1, 0, 2)).reshape(T_ENC * N, H)
    hs_e, hd_e = _mm2(ysT, enc["gsrc_W"], enc["gsrc_b"],
                      enc["gdst_W"], enc["gdst_b"], tm=400)
    HEADS_E = 4
    rst_e = _gat_sc(hs_e, hd_e, enc["attn"], src_pad, seg_pad, own_pad,
                    T_ENC, HEADS_E)
    fcW_e = jnp.tile(enc["fc_W"], (HEADS_E, 1)) / HEADS_E
    fcb_e = enc["fc_b"] + (enc["gbias"].reshape(HEADS_E, H).mean(0)
                           @ enc["fc_W"])
    enc_out_T = _mm(rst_e, fcW_e, fcb_e, tm=400)
    enc_out = jnp.transpose(enc_out_T.reshape(T_ENC, N, OUT), (1, 0, 2))

    fcW_d = dec["fc_W"]
    fcb_d = dec["fc_b"] + dec["gbias"] @ dec["fc_W"]

    hidden = h_enc[0:1, :]
    cell = c_enc[0:1, :]
    current = enc_out[:, T_ENC - 1, :]
    outs = []
    for t in range(TRG_LEN):
        inp = jnp.concatenate([current, src2[:, t, :]], axis=1)
        ys_d, hidden, cell = _lstm_scan(inp[:, None, :], dec_w1, dec_b1,
                                        dec_whhT, hidden, cell, chunk=500)
        feat = ys_d[:, 0, :]
        hs_d, hd_d = _mm2(feat, dec["gsrc_W"], dec["gsrc_b"],
                          dec["gdst_W"], dec["gdst_b"], tm=400)
        rst_d = _gat_sc(hs_d, hd_d, dec["attn"], src_pad, seg_pad,
                        own_pad, 1, 1)
        o = _mm(rst_d, fcW_d, fcb_d, tm=400)
        outs.append(o)
        current = o

    outputs = jnp.stack(outs, axis=1)
    prec = enc_out[:, 0:1, :]
    return jnp.concatenate([prec, outputs], axis=1)
